# Initial kernel scaffold; baseline (speedup 1.0000x reference)
import jax
import jax.numpy as jnp
from jax.experimental import pallas as pl


def kernel(x_paper, x_author, x_venue, x_topic, edge_index_cites, edge_index_authored_by, edge_index_published_in, edge_index_discusses, params):
    raise NotImplementedError("write your pallas kernel here")


# speedup vs baseline: 7.1478x; 7.1478x over previous
import functools
import jax
import jax.numpy as jnp
from jax import lax
from jax.experimental import pallas as pl
from jax.experimental.pallas import tpu as pltpu
from jax.experimental.pallas import tpu_sc as plsc

_HID = 128
_HEADS = 4
_NTYPES = ('paper', 'author', 'venue', 'topic')
_NN = {'paper': 50000, 'author': 50000, 'venue': 10000, 'topic': 10000}
_RELS = (('cites', 'paper', 'paper', 400000),
         ('authored_by', 'paper', 'author', 200000),
         ('published_in', 'paper', 'venue', 50000),
         ('discusses', 'paper', 'topic', 100000))
_NC, _NS = 2, 16
_NW = _NC * _NS
_MB = 128
_NPAD = {'paper': 51200, 'author': 51200, 'venue': 10240, 'topic': 10240}
_BUCKET = 12800
_ROLES = {
    'paper': [('cites', 'src'), ('authored_by', 'src'), ('published_in', 'src'),
              ('discusses', 'src'), ('cites', 'dst')],
    'author': [('authored_by', 'dst')],
    'venue': [('published_in', 'dst')],
    'topic': [('discusses', 'dst')],
}
_REL_FULLKEY = {'cites': 'paper__cites__paper',
                'authored_by': 'paper__authored_by__author',
                'published_in': 'paper__published_in__venue',
                'discusses': 'paper__discusses__topic'}


def _epad(e):
    blk = _NW * _MB
    return ((e + blk - 1) // blk) * blk


def _tc_proj(n_rows, n_roles, merge, relu_in, n_rows_pad=0):
    bn = 1000
    grid = n_rows // bn
    w16 = 16 * n_roles

    def body(*refs):
        i = pl.program_id(0)
        if merge:
            o0, o1, w, b = refs[:4]
            x = o0[...] + o1[...]
            if relu_in:
                x = jnp.maximum(x, 0.0)
        else:
            xr, w, b = refs[:3]
            x = xr[...]
        nin = 4 if merge else 3
        xp = jnp.dot(x, w[...], preferred_element_type=jnp.float32) + b[...][0]
        if n_roles:
            lin = refs[nin]
            xp_ref = refs[nin + 1]
            a_refs = refs[nin + 2:nin + 2 + n_roles]
            amax_ref = refs[nin + 2 + n_roles]
            xp_ref[...] = xp
            av = jnp.dot(xp, lin[...], preferred_element_type=jnp.float32)
            for r in range(n_roles):
                a_refs[r][...] = av[:, 16 * r:16 * r + 16]
            cur = jnp.broadcast_to(jnp.max(av, axis=0, keepdims=True),
                                   (8, w16))

            @pl.when(i == 0)
            def _():
                amax_ref[...] = cur

            @pl.when(i != 0)
            def _():
                amax_ref[...] = jnp.maximum(amax_ref[...], cur)
        else:
            xp_ref = refs[nin]
            xp_ref[...] = xp

    row_spec = pl.BlockSpec((bn, _HID), lambda i: (i, 0))
    full = lambda shape: pl.BlockSpec(shape, lambda i: (0, 0))
    in_specs = ([row_spec, row_spec] if merge else [row_spec])
    in_specs += [full((_HID, _HID)), full((8, _HID))]
    out_shapes = [jax.ShapeDtypeStruct((n_rows, _HID), jnp.float32)]
    out_specs = [row_spec]
    if n_roles:
        in_specs += [full((_HID, w16))]
        out_shapes += [jax.ShapeDtypeStruct((n_rows_pad, 16), jnp.float32)
                       for _ in range(n_roles)]
        out_specs += [pl.BlockSpec((bn, 16), lambda i: (i, 0))
                      for _ in range(n_roles)]
        out_shapes += [jax.ShapeDtypeStruct((8, w16), jnp.float32)]
        out_specs += [full((8, w16))]

    return pl.pallas_call(
        body, grid=(grid,), in_specs=in_specs,
        out_specs=out_specs, out_shape=out_shapes)


_SC_PARAMS = pltpu.CompilerParams(use_tc_tiling_on_sc=False)

_MESH = functools.partial(plsc.VectorSubcoreMesh,
                          core_axis_name='c', subcore_axis_name='s',
                          num_cores=_NC, num_subcores=_NS)


def _leaky(s):
    return jnp.maximum(s, 0.0) + 0.2 * jnp.minimum(s, 0.0)


def _sc_p1(epads, e_reals, npads):
    nrel = len(_RELS)

    out_type = []
    for r in range(nrel):
        out_type.append(jax.ShapeDtypeStruct((epads[r], 16), jnp.float32))
        out_type.append(jax.ShapeDtypeStruct((npads[r], 16), jnp.float32))
        out_type.append(jax.ShapeDtypeStruct((npads[r], 16), jnp.float32))

    scratch = [
        pltpu.VMEM((_MB,), jnp.int32),
        pltpu.VMEM((_MB,), jnp.int32),
        pltpu.VMEM((_MB, 16), jnp.float32),
        pltpu.VMEM((_MB, 16), jnp.float32),
        pltpu.VMEM((_MB, 16), jnp.float32),
        pltpu.VMEM((16,), jnp.float32),
        pltpu.SemaphoreType.DMA,
    ] + [pltpu.VMEM_SHARED((npads[r], 16), jnp.float32) for r in range(nrel)]

    def body(*refs):
        zero_den = refs[0]
        ins = refs[1:1 + 5 * nrel]
        outs = refs[1 + 5 * nrel:1 + 8 * nrel]
        (src_v, dst_v, asrc_v, adst_v, ex_v, mv, sem) = refs[1 + 8 * nrel:
                                                             8 + 8 * nrel]
        den_sps = refs[8 + 8 * nrel:]
        cid = lax.axis_index('c')
        sid = lax.axis_index('s')
        wid = sid * _NC + cid

        for r in range(nrel):
            src_h, dst_h, a_src_h, a_dst_h, mvec_h = ins[5 * r:5 * r + 5]
            ex_h, den0_h, den1_h = outs[3 * r:3 * r + 3]
            den_sp = den_sps[r]
            npad, epad, ereal = npads[r], epads[r], e_reals[r]
            rows = npad // _NS
            pltpu.sync_copy(zero_den.at[pl.ds(0, rows)],
                            den_sp.at[pl.ds(sid * rows, rows)])
            plsc.subcore_barrier()
            pltpu.sync_copy(mvec_h, mv)
            m16 = mv[...]
            ew = epad // _NW
            nb = ew // _MB
            base = wid * ew

            def batch(t, carry):
                off = base + t * _MB
                pltpu.sync_copy(src_h.at[pl.ds(off, _MB)], src_v)
                pltpu.sync_copy(dst_h.at[pl.ds(off, _MB)], dst_v)
                pltpu.async_copy(a_src_h.at[src_v], asrc_v, sem).wait()
                pltpu.async_copy(a_dst_h.at[dst_v], adst_v, sem).wait()
                for j in range(_MB):
                    s = asrc_v[j, :] + adst_v[j, :]
                    ex_v[j, :] = jnp.exp(_leaky(s) - m16)
                pltpu.sync_copy(ex_v, ex_h.at[pl.ds(off, _MB)])
                pltpu.sync_copy(ex_v, den_sp.at[dst_v], add=True)
                return carry

            lax.fori_loop(0, nb, batch, 0)
            plsc.subcore_barrier()

            @pl.when(cid == 0)
            def _():
                pltpu.sync_copy(den_sp.at[pl.ds(sid * rows, rows)],
                                den0_h.at[pl.ds(sid * rows, rows)])

            @pl.when(cid == 1)
            def _():
                pltpu.sync_copy(den_sp.at[pl.ds(sid * rows, rows)],
                                den1_h.at[pl.ds(sid * rows, rows)])
            plsc.subcore_barrier()

    return pl.kernel(body, out_type=tuple(out_type), mesh=_MESH(),
                     scratch_types=scratch, compiler_params=_SC_PARAMS)


def _sc_p3(epads, npads):
    nrel = len(_RELS)
    out_type = []
    nbks = []
    for r in range(nrel):
        out_type.append(jax.ShapeDtypeStruct((npads[r] * 8, 16), jnp.float32))
        out_type.append(jax.ShapeDtypeStruct((npads[r] * 8, 16), jnp.float32))
        nbks.append(npads[r] // _BUCKET if npads[r] >= _BUCKET else 1)

    scratch = [
        pltpu.VMEM((_MB,), jnp.int32),
    ] + [pltpu.VMEM((_MB,), jnp.int32) for _ in range(8)] + [
    ] + [pltpu.VMEM((_MB,), jnp.int32) for _ in range(8)] + [
        pltpu.VMEM((_MB, 16), jnp.float32),
        pltpu.VMEM((_MB, 16), jnp.float32),
        pltpu.VMEM((_MB, 16), jnp.float32),
        pltpu.VMEM((_MB, 16), jnp.float32),
        pltpu.VMEM((_MB * 8, 16), jnp.float32),
        pltpu.SemaphoreType.DMA,
        pltpu.VMEM_SHARED(((_BUCKET + 1) * 8, 16), jnp.float32),
    ]
    n_in = [5 + nbks[r] for r in range(nrel)]

    def body(*refs):
        zero_acc = refs[0]
        xp2 = refs[1]
        ins = refs[2:2 + sum(n_in)]
        outs = refs[2 + sum(n_in):2 + sum(n_in) + 2 * nrel]
        sc_refs = refs[2 + sum(n_in) + 2 * nrel:]
        dst_v = sc_refs[0]
        eidx_vs = sc_refs[1:9]
        didx_vs = sc_refs[9:17]
        (ex_v, d0_v, d1_v, w_v, xp_v, sem, acc) = sc_refs[17:]
        cid = lax.axis_index('c')
        sid = lax.axis_index('s')
        wid = sid * _NC + cid

        ioff = 0
        for r in range(nrel):
            dst_h, ex_h, den0_h, den1_h, eidx_h = ins[ioff:ioff + 5]
            didx_hs = ins[ioff + 5:ioff + n_in[r]]
            ioff += n_in[r]
            o0_h, o1_h = outs[2 * r:2 * r + 2]
            npad, epad = npads[r], epads[r]
            nbuckets = nbks[r]
            brows = npad // nbuckets
            rows_pb = brows // _NS
            ew = epad // _NW
            nb = ew // _MB
            base = wid * ew

            for bkt in range(nbuckets):
                lo = bkt * brows
                didx_h = didx_hs[bkt]
                pltpu.sync_copy(zero_acc.at[pl.ds(0, rows_pb * 8)],
                                acc.at[pl.ds(sid * rows_pb * 8, rows_pb * 8)])
                plsc.subcore_barrier()

                def batch(t, carry):
                    off = base + t * _MB
                    pltpu.sync_copy(dst_h.at[pl.ds(off, _MB)], dst_v)
                    pltpu.sync_copy(ex_h.at[pl.ds(off, _MB)], ex_v)
                    for k in range(8):
                        pltpu.sync_copy(eidx_h.at[off // 16 + k], eidx_vs[k])
                        pltpu.sync_copy(didx_h.at[off // 16 + k], didx_vs[k])
                    pltpu.async_copy(den0_h.at[dst_v], d0_v, sem).wait()
                    pltpu.async_copy(den1_h.at[dst_v], d1_v, sem).wait()
                    for k in range(8):
                        pltpu.async_copy(xp2.at[eidx_vs[k]],
                                         xp_v.at[pl.ds(_MB * k, _MB)],
                                         sem).wait()

                    def wrow(j, cw):
                        w_v[j, :] = ex_v[j, :] / (d0_v[j, :] + d1_v[j, :]
                                                  + 1e-16)
                        return cw

                    lax.fori_loop(0, _MB, wrow, 0)

                    def edge(e, c2):
                        w16 = w_v[e, :]
                        r0 = 8 * e
                        for h in range(4):
                            wb = w16.at[jnp.full((16,), h, jnp.int32)].get(
                                mode='promise_in_bounds')
                            for j in (2 * h, 2 * h + 1):
                                xp_v[r0 + j, :] = xp_v[r0 + j, :] * wb
                        return c2

                    lax.fori_loop(0, _MB, edge, 0)
                    for k in range(8):
                        pltpu.sync_copy(xp_v.at[pl.ds(_MB * k, _MB)],
                                        acc.at[didx_vs[k]], add=True)
                    return carry

                lax.fori_loop(0, nb, batch, 0)
                plsc.subcore_barrier()

                @pl.when(cid == 0)
                def _():
                    pltpu.sync_copy(
                        acc.at[pl.ds(sid * rows_pb * 8, rows_pb * 8)],
                        o0_h.at[pl.ds((lo + sid * rows_pb) * 8,
                                      rows_pb * 8)])

                @pl.when(cid == 1)
                def _():
                    pltpu.sync_copy(
                        acc.at[pl.ds(sid * rows_pb * 8, rows_pb * 8)],
                        o1_h.at[pl.ds((lo + sid * rows_pb) * 8,
                                      rows_pb * 8)])
                plsc.subcore_barrier()

    return pl.kernel(body, out_type=tuple(out_type), mesh=_MESH(),
                     scratch_types=scratch, compiler_params=_SC_PARAMS)


def _build_lin(lp):
    lins = {}
    for nt in _NTYPES:
        roles = _ROLES[nt]
        cols = jnp.zeros((_HID, 16 * len(roles)), jnp.float32)
        for ridx, (rel, role) in enumerate(roles):
            key = _REL_FULLKEY[rel]
            vec = lp['lin_src'][key] if role == 'src' else lp['lin_dst'][key]
            for h in range(_HEADS):
                cols = cols.at[32 * h:32 * h + 32, 16 * ridx + h].set(vec[h])
        lins[nt] = cols
    return lins


def kernel(x_paper, x_author, x_venue, x_topic, edge_index_cites,
           edge_index_authored_by, edge_index_published_in,
           edge_index_discusses, params):
    x_in = {'paper': x_paper, 'author': x_author, 'venue': x_venue,
            'topic': x_topic}
    e_idx = {'cites': edge_index_cites, 'authored_by': edge_index_authored_by,
             'published_in': edge_index_published_in,
             'discusses': edge_index_discusses}

    epads = [_epad(e) for (_, _, _, e) in _RELS]
    e_reals = [e for (_, _, _, e) in _RELS]
    npads = [_NPAD[dt] for (_, _, dt, _) in _RELS]
    srcs, dsts = [], []
    for ri, (rel, st, dt, e) in enumerate(_RELS):
        ei = e_idx[rel]
        pad = epads[ri] - e
        srcs.append(jnp.pad(ei[0], (0, pad)))
        dsts.append(jnp.pad(ei[1], (0, pad), constant_values=npads[ri] - 1))

    zero_den = jnp.zeros((_NPAD['paper'] // _NS, 16), jnp.float32)
    zero_acc = jnp.zeros((_BUCKET // _NS * 8, 16), jnp.float32)

    j8 = jnp.arange(8, dtype=jnp.int32)
    eidx8s, didx8s = [], []
    for ri in range(len(_RELS)):
        npad = npads[ri]
        eidx8s.append((8 * srcs[ri][:, None] + j8).reshape(-1, 128))
        nbk = npad // _BUCKET if npad >= _BUCKET else 1
        brows = npad // nbk
        dlist = []
        for bkt in range(nbk):
            rel = dsts[ri] - bkt * brows
            rel = jnp.where((rel >= 0) & (rel < brows), rel, _BUCKET)
            dlist.append((8 * rel[:, None] + j8).reshape(-1, 128))
        didx8s.append(dlist)

    p1 = _sc_p1(epads, e_reals, npads)
    p3 = _sc_p3(epads, npads)

    prev = None
    for lp in params['layers']:
        lins = _build_lin(lp)
        xp, a_arrs, amax = {}, {}, {}
        for nt in _NTYPES:
            n_roles = len(_ROLES[nt])
            tc = _tc_proj(_NN[nt], n_roles, merge=prev is not None,
                          relu_in=True, n_rows_pad=_NPAD[nt])
            b8 = jnp.tile(lp['proj_b'][nt][None, :], (8, 1))
            if prev is None:
                res = tc(x_in[nt], lp['proj_w'][nt], b8, lins[nt])
            else:
                o0, o1 = prev[nt]
                res = tc(o0, o1, lp['proj_w'][nt], b8, lins[nt])
            xp[nt] = res[0]
            a_arrs[nt] = res[1:1 + n_roles]
            amax[nt] = res[1 + n_roles]

        p1_ins = [zero_den]
        for ri, (rel, st, dt, e) in enumerate(_RELS):
            src_role = _ROLES[st].index((rel, 'src'))
            dst_role = _ROLES[dt].index((rel, 'dst'))
            a_src = a_arrs[st][src_role]
            a_dst = a_arrs[dt][dst_role]
            msrc = amax[st][0, 16 * src_role:16 * src_role + 4]
            mdst = amax[dt][0, 16 * dst_role:16 * dst_role + 4]
            m4 = _leaky(msrc + mdst)
            mvec = jnp.concatenate([m4, jnp.full((12,), 1e30, jnp.float32)])
            p1_ins += [srcs[ri], dsts[ri], a_src, a_dst, mvec]
        p1_out = p1(*p1_ins)

        p3_ins = [zero_acc, xp['paper'].reshape(_NN['paper'] * 8, 16)]
        for ri in range(len(_RELS)):
            ex, den0, den1 = p1_out[3 * ri:3 * ri + 3]
            p3_ins += [dsts[ri], ex, den0, den1, eidx8s[ri]] + didx8s[ri]
        p3_out = p3(*p3_ins)

        prev = {}
        for ri, (rel, st, dt, e) in enumerate(_RELS):
            npad = npads[ri]
            prev[dt] = (p3_out[2 * ri].reshape(npad, _HID),
                        p3_out[2 * ri + 1].reshape(npad, _HID))

    outs = []
    for nt in _NTYPES:
        tc = _tc_proj(_NN[nt], 0, merge=True, relu_in=True)
        b8 = jnp.tile(params['out_b'][nt][None, :], (8, 1))
        o0, o1 = prev[nt]
        outs.append(tc(o0, o1, params['out_w'][nt], b8)[0])
    return tuple(outs)


# speedup vs baseline: 10.2027x; 1.4274x over previous
import functools
import jax
import jax.numpy as jnp
from jax import lax
from jax.experimental import pallas as pl
from jax.experimental.pallas import tpu as pltpu
from jax.experimental.pallas import tpu_sc as plsc

_HID = 128
_HEADS = 4
_NTYPES = ('paper', 'author', 'venue', 'topic')
_NN = {'paper': 50000, 'author': 50000, 'venue': 10000, 'topic': 10000}
_RELS = (('cites', 'paper', 'paper', 400000),
         ('authored_by', 'paper', 'author', 200000),
         ('published_in', 'paper', 'venue', 50000),
         ('discusses', 'paper', 'topic', 100000))
_NC, _NS = 2, 16
_NW = _NC * _NS
_MB = 128
_NPAD = {'paper': 51200, 'author': 51200, 'venue': 10240, 'topic': 10240}
_BUCKET = 12800
_ROLES = {
    'paper': [('cites', 'src'), ('authored_by', 'src'), ('published_in', 'src'),
              ('discusses', 'src'), ('cites', 'dst')],
    'author': [('authored_by', 'dst')],
    'venue': [('published_in', 'dst')],
    'topic': [('discusses', 'dst')],
}
_REL_FULLKEY = {'cites': 'paper__cites__paper',
                'authored_by': 'paper__authored_by__author',
                'published_in': 'paper__published_in__venue',
                'discusses': 'paper__discusses__topic'}


def _epad(e):
    blk = _NW * _MB
    return ((e + blk - 1) // blk) * blk


def _tc_proj(n_rows, n_roles, merge, relu_in, n_rows_pad=0):
    bn = 1000
    grid = n_rows // bn
    w16 = 16 * n_roles

    def body(*refs):
        i = pl.program_id(0)
        if merge:
            o0, o1, w, b = refs[:4]
            x = o0[...] + o1[...]
            if relu_in:
                x = jnp.maximum(x, 0.0)
        else:
            xr, w, b = refs[:3]
            x = xr[...]
        nin = 4 if merge else 3
        xp = jnp.dot(x, w[...], preferred_element_type=jnp.float32) + b[...][0]
        if n_roles:
            lin = refs[nin]
            xp_ref = refs[nin + 1]
            a_refs = refs[nin + 2:nin + 2 + n_roles]
            amax_ref = refs[nin + 2 + n_roles]
            xp_ref[...] = xp
            av = jnp.dot(xp, lin[...], preferred_element_type=jnp.float32)
            for r in range(n_roles):
                a_refs[r][...] = av[:, 16 * r:16 * r + 16]
            cur = jnp.broadcast_to(jnp.max(av, axis=0, keepdims=True),
                                   (8, w16))

            @pl.when(i == 0)
            def _():
                amax_ref[...] = cur

            @pl.when(i != 0)
            def _():
                amax_ref[...] = jnp.maximum(amax_ref[...], cur)
        else:
            xp_ref = refs[nin]
            xp_ref[...] = xp

    row_spec = pl.BlockSpec((bn, _HID), lambda i: (i, 0))
    full = lambda shape: pl.BlockSpec(shape, lambda i: (0, 0))
    in_specs = ([row_spec, row_spec] if merge else [row_spec])
    in_specs += [full((_HID, _HID)), full((8, _HID))]
    out_shapes = [jax.ShapeDtypeStruct((n_rows, _HID), jnp.float32)]
    out_specs = [row_spec]
    if n_roles:
        in_specs += [full((_HID, w16))]
        out_shapes += [jax.ShapeDtypeStruct((n_rows_pad, 16), jnp.float32)
                       for _ in range(n_roles)]
        out_specs += [pl.BlockSpec((bn, 16), lambda i: (i, 0))
                      for _ in range(n_roles)]
        out_shapes += [jax.ShapeDtypeStruct((8, w16), jnp.float32)]
        out_specs += [full((8, w16))]

    return pl.pallas_call(
        body, grid=(grid,), in_specs=in_specs,
        out_specs=out_specs, out_shape=out_shapes)


_SC_PARAMS = pltpu.CompilerParams(use_tc_tiling_on_sc=False)

_MESH = functools.partial(plsc.VectorSubcoreMesh,
                          core_axis_name='c', subcore_axis_name='s',
                          num_cores=_NC, num_subcores=_NS)


def _leaky(s):
    return jnp.maximum(s, 0.0) + 0.2 * jnp.minimum(s, 0.0)


def _sc_p1(epads, e_reals, npads):
    nrel = len(_RELS)

    out_type = []
    for r in range(nrel):
        out_type.append(jax.ShapeDtypeStruct((epads[r], 16), jnp.float32))
        out_type.append(jax.ShapeDtypeStruct((npads[r], 16), jnp.float32))
        out_type.append(jax.ShapeDtypeStruct((npads[r], 16), jnp.float32))

    scratch = [
        pltpu.VMEM((_MB,), jnp.int32),
        pltpu.VMEM((_MB,), jnp.int32),
        pltpu.VMEM((_MB, 16), jnp.float32),
        pltpu.VMEM((_MB, 16), jnp.float32),
        pltpu.VMEM((_MB, 16), jnp.float32),
        pltpu.VMEM((16,), jnp.float32),
        pltpu.SemaphoreType.DMA,
    ] + [pltpu.VMEM_SHARED((npads[r], 16), jnp.float32) for r in range(nrel)]

    def body(*refs):
        zero_den = refs[0]
        ins = refs[1:1 + 5 * nrel]
        outs = refs[1 + 5 * nrel:1 + 8 * nrel]
        (src_v, dst_v, asrc_v, adst_v, ex_v, mv, sem) = refs[1 + 8 * nrel:
                                                             8 + 8 * nrel]
        den_sps = refs[8 + 8 * nrel:]
        cid = lax.axis_index('c')
        sid = lax.axis_index('s')
        wid = sid * _NC + cid

        for r in range(nrel):
            src_h, dst_h, a_src_h, a_dst_h, mvec_h = ins[5 * r:5 * r + 5]
            ex_h, den0_h, den1_h = outs[3 * r:3 * r + 3]
            den_sp = den_sps[r]
            npad, epad, ereal = npads[r], epads[r], e_reals[r]
            rows = npad // _NS
            pltpu.sync_copy(zero_den.at[pl.ds(0, rows)],
                            den_sp.at[pl.ds(sid * rows, rows)])
            plsc.subcore_barrier()
            pltpu.sync_copy(mvec_h, mv)
            m16 = mv[...]
            ew = epad // _NW
            nb = ew // _MB
            base = wid * ew

            def batch(t, carry):
                off = base + t * _MB
                pltpu.sync_copy(src_h.at[pl.ds(off, _MB)], src_v)
                pltpu.sync_copy(dst_h.at[pl.ds(off, _MB)], dst_v)
                pltpu.async_copy(a_src_h.at[src_v], asrc_v, sem).wait()
                pltpu.async_copy(a_dst_h.at[dst_v], adst_v, sem).wait()
                for j in range(_MB):
                    s = asrc_v[j, :] + adst_v[j, :]
                    ex_v[j, :] = jnp.exp(_leaky(s) - m16)
                pltpu.sync_copy(ex_v, ex_h.at[pl.ds(off, _MB)])
                pltpu.sync_copy(ex_v, den_sp.at[dst_v], add=True)
                return carry

            lax.fori_loop(0, nb, batch, 0)
            plsc.subcore_barrier()

            @pl.when(cid == 0)
            def _():
                pltpu.sync_copy(den_sp.at[pl.ds(sid * rows, rows)],
                                den0_h.at[pl.ds(sid * rows, rows)])

            @pl.when(cid == 1)
            def _():
                pltpu.sync_copy(den_sp.at[pl.ds(sid * rows, rows)],
                                den1_h.at[pl.ds(sid * rows, rows)])
            plsc.subcore_barrier()

    return pl.kernel(body, out_type=tuple(out_type), mesh=_MESH(),
                     scratch_types=scratch, compiler_params=_SC_PARAMS)


def _sc_p3(epads, npads):
    nrel = len(_RELS)
    out_type = []
    nbks = []
    for r in range(nrel):
        out_type.append(jax.ShapeDtypeStruct((npads[r] * 8, 16), jnp.float32))
        out_type.append(jax.ShapeDtypeStruct((npads[r] * 8, 16), jnp.float32))
        nbks.append(npads[r] // _BUCKET if npads[r] >= _BUCKET else 1)

    scratch = [
        pltpu.VMEM((_MB,), jnp.int32),
    ] + [pltpu.VMEM((_MB,), jnp.int32) for _ in range(8)] + [
    ] + [pltpu.VMEM((_MB,), jnp.int32) for _ in range(8)] + [
        pltpu.VMEM((_MB, 16), jnp.float32),
        pltpu.VMEM((_MB, 16), jnp.float32),
        pltpu.VMEM((_MB, 16), jnp.float32),
        pltpu.VMEM((_MB, 16), jnp.float32),
        pltpu.VMEM((_MB * 8, 16), jnp.float32),
        pltpu.SemaphoreType.DMA,
        pltpu.SemaphoreType.DMA,
        pltpu.VMEM_SHARED(((_BUCKET + 1) * 8, 16), jnp.float32),
    ]
    n_in = [5 + nbks[r] for r in range(nrel)]

    def body(*refs):
        zero_acc = refs[0]
        xp2 = refs[1]
        ins = refs[2:2 + sum(n_in)]
        outs = refs[2 + sum(n_in):2 + sum(n_in) + 2 * nrel]
        sc_refs = refs[2 + sum(n_in) + 2 * nrel:]
        dst_v = sc_refs[0]
        eidx_vs = sc_refs[1:9]
        didx_vs = sc_refs[9:17]
        (ex_v, d0_v, d1_v, w_v, xp_v, sem, sem2, acc) = sc_refs[17:]
        cid = lax.axis_index('c')
        sid = lax.axis_index('s')
        wid = sid * _NC + cid

        ioff = 0
        for r in range(nrel):
            dst_h, ex_h, den0_h, den1_h, eidx_h = ins[ioff:ioff + 5]
            didx_hs = ins[ioff + 5:ioff + n_in[r]]
            ioff += n_in[r]
            o0_h, o1_h = outs[2 * r:2 * r + 2]
            npad, epad = npads[r], epads[r]
            nbuckets = nbks[r]
            brows = npad // nbuckets
            rows_pb = brows // _NS
            ew = epad // _NW
            nb = ew // _MB
            base = wid * ew

            for bkt in range(nbuckets):
                lo = bkt * brows
                didx_h = didx_hs[bkt]
                pltpu.sync_copy(zero_acc.at[pl.ds(0, rows_pb * 8)],
                                acc.at[pl.ds(sid * rows_pb * 8, rows_pb * 8)])
                plsc.subcore_barrier()

                def batch(t, carry):
                    off = base + t * _MB
                    pltpu.sync_copy(dst_h.at[pl.ds(off, _MB)], dst_v)
                    pltpu.sync_copy(ex_h.at[pl.ds(off, _MB)], ex_v)
                    for k in range(8):
                        pltpu.sync_copy(eidx_h.at[off // 16 + k], eidx_vs[k])
                        pltpu.sync_copy(didx_h.at[off // 16 + k], didx_vs[k])
                    cpd0 = pltpu.async_copy(den0_h.at[dst_v], d0_v, sem2)
                    cpd1 = pltpu.async_copy(den1_h.at[dst_v], d1_v, sem2)
                    cps = [pltpu.async_copy(xp2.at[eidx_vs[k]],
                                            xp_v.at[pl.ds(_MB * k, _MB)],
                                            sem)
                           for k in range(8)]
                    cpd0.wait()
                    cpd1.wait()

                    def wrow(j, cw):
                        w_v[j, :] = ex_v[j, :] / (d0_v[j, :] + d1_v[j, :]
                                                  + 1e-16)
                        return cw

                    lax.fori_loop(0, _MB, wrow, 0)
                    for cp in cps:
                        cp.wait()

                    def edge(e, c2):
                        w16 = w_v[e, :]
                        r0 = 8 * e
                        for h in range(4):
                            wb = w16.at[jnp.full((16,), h, jnp.int32)].get(
                                mode='promise_in_bounds')
                            for j in (2 * h, 2 * h + 1):
                                xp_v[r0 + j, :] = xp_v[r0 + j, :] * wb
                        return c2

                    lax.fori_loop(0, _MB, edge, 0)
                    scs = [pltpu.async_copy(xp_v.at[pl.ds(_MB * k, _MB)],
                                            acc.at[didx_vs[k]], sem2,
                                            add=True)
                           for k in range(8)]
                    for sc in scs:
                        sc.wait()
                    return carry

                lax.fori_loop(0, nb, batch, 0)
                plsc.subcore_barrier()

                @pl.when(cid == 0)
                def _():
                    pltpu.sync_copy(
                        acc.at[pl.ds(sid * rows_pb * 8, rows_pb * 8)],
                        o0_h.at[pl.ds((lo + sid * rows_pb) * 8,
                                      rows_pb * 8)])

                @pl.when(cid == 1)
                def _():
                    pltpu.sync_copy(
                        acc.at[pl.ds(sid * rows_pb * 8, rows_pb * 8)],
                        o1_h.at[pl.ds((lo + sid * rows_pb) * 8,
                                      rows_pb * 8)])
                plsc.subcore_barrier()

    return pl.kernel(body, out_type=tuple(out_type), mesh=_MESH(),
                     scratch_types=scratch, compiler_params=_SC_PARAMS)


def _build_lin(lp):
    lins = {}
    for nt in _NTYPES:
        roles = _ROLES[nt]
        cols = jnp.zeros((_HID, 16 * len(roles)), jnp.float32)
        for ridx, (rel, role) in enumerate(roles):
            key = _REL_FULLKEY[rel]
            vec = lp['lin_src'][key] if role == 'src' else lp['lin_dst'][key]
            for h in range(_HEADS):
                cols = cols.at[32 * h:32 * h + 32, 16 * ridx + h].set(vec[h])
        lins[nt] = cols
    return lins


def kernel(x_paper, x_author, x_venue, x_topic, edge_index_cites,
           edge_index_authored_by, edge_index_published_in,
           edge_index_discusses, params):
    x_in = {'paper': x_paper, 'author': x_author, 'venue': x_venue,
            'topic': x_topic}
    e_idx = {'cites': edge_index_cites, 'authored_by': edge_index_authored_by,
             'published_in': edge_index_published_in,
             'discusses': edge_index_discusses}

    epads = [_epad(e) for (_, _, _, e) in _RELS]
    e_reals = [e for (_, _, _, e) in _RELS]
    npads = [_NPAD[dt] for (_, _, dt, _) in _RELS]
    srcs, dsts = [], []
    for ri, (rel, st, dt, e) in enumerate(_RELS):
        ei = e_idx[rel]
        pad = epads[ri] - e
        srcs.append(jnp.pad(ei[0], (0, pad)))
        dsts.append(jnp.pad(ei[1], (0, pad), constant_values=npads[ri] - 1))

    zero_den = jnp.zeros((_NPAD['paper'] // _NS, 16), jnp.float32)
    zero_acc = jnp.zeros((_BUCKET // _NS * 8, 16), jnp.float32)

    j8 = jnp.arange(8, dtype=jnp.int32)
    eidx8s, didx8s = [], []
    for ri in range(len(_RELS)):
        npad = npads[ri]
        eidx8s.append((8 * srcs[ri][:, None] + j8).reshape(-1, 128))
        nbk = npad // _BUCKET if npad >= _BUCKET else 1
        brows = npad // nbk
        dlist = []
        for bkt in range(nbk):
            rel = dsts[ri] - bkt * brows
            rel = jnp.where((rel >= 0) & (rel < brows), rel, _BUCKET)
            dlist.append((8 * rel[:, None] + j8).reshape(-1, 128))
        didx8s.append(dlist)

    p1 = _sc_p1(epads, e_reals, npads)
    p3 = _sc_p3(epads, npads)

    prev = None
    for lp in params['layers']:
        lins = _build_lin(lp)
        xp, a_arrs, amax = {}, {}, {}
        for nt in _NTYPES:
            n_roles = len(_ROLES[nt])
            tc = _tc_proj(_NN[nt], n_roles, merge=prev is not None,
                          relu_in=True, n_rows_pad=_NPAD[nt])
            b8 = jnp.tile(lp['proj_b'][nt][None, :], (8, 1))
            if prev is None:
                res = tc(x_in[nt], lp['proj_w'][nt], b8, lins[nt])
            else:
                o0, o1 = prev[nt]
                res = tc(o0, o1, lp['proj_w'][nt], b8, lins[nt])
            xp[nt] = res[0]
            a_arrs[nt] = res[1:1 + n_roles]
            amax[nt] = res[1 + n_roles]

        p1_ins = [zero_den]
        for ri, (rel, st, dt, e) in enumerate(_RELS):
            src_role = _ROLES[st].index((rel, 'src'))
            dst_role = _ROLES[dt].index((rel, 'dst'))
            a_src = a_arrs[st][src_role]
            a_dst = a_arrs[dt][dst_role]
            msrc = amax[st][0, 16 * src_role:16 * src_role + 4]
            mdst = amax[dt][0, 16 * dst_role:16 * dst_role + 4]
            m4 = _leaky(msrc + mdst)
            mvec = jnp.concatenate([m4, jnp.full((12,), 1e30, jnp.float32)])
            p1_ins += [srcs[ri], dsts[ri], a_src, a_dst, mvec]
        p1_out = p1(*p1_ins)

        p3_ins = [zero_acc, xp['paper'].reshape(_NN['paper'] * 8, 16)]
        for ri in range(len(_RELS)):
            ex, den0, den1 = p1_out[3 * ri:3 * ri + 3]
            p3_ins += [dsts[ri], ex, den0, den1, eidx8s[ri]] + didx8s[ri]
        p3_out = p3(*p3_ins)

        prev = {}
        for ri, (rel, st, dt, e) in enumerate(_RELS):
            npad = npads[ri]
            prev[dt] = (p3_out[2 * ri].reshape(npad, _HID),
                        p3_out[2 * ri + 1].reshape(npad, _HID))

    outs = []
    for nt in _NTYPES:
        tc = _tc_proj(_NN[nt], 0, merge=True, relu_in=True)
        b8 = jnp.tile(params['out_b'][nt][None, :], (8, 1))
        o0, o1 = prev[nt]
        outs.append(tc(o0, o1, params['out_w'][nt], b8)[0])
    return tuple(outs)


# speedup vs baseline: 10.4064x; 1.0200x over previous
import functools
import jax
import jax.numpy as jnp
from jax import lax
from jax.experimental import pallas as pl
from jax.experimental.pallas import tpu as pltpu
from jax.experimental.pallas import tpu_sc as plsc

_HID = 128
_HEADS = 4
_NTYPES = ('paper', 'author', 'venue', 'topic')
_NN = {'paper': 50000, 'author': 50000, 'venue': 10000, 'topic': 10000}
_RELS = (('cites', 'paper', 'paper', 400000),
         ('authored_by', 'paper', 'author', 200000),
         ('published_in', 'paper', 'venue', 50000),
         ('discusses', 'paper', 'topic', 100000))
_NC, _NS = 2, 16
_NW = _NC * _NS
_MB = 128
_NPAD = {'paper': 51200, 'author': 51200, 'venue': 10240, 'topic': 10240}
_BUCKET = 12800
_ROLES = {
    'paper': [('cites', 'src'), ('authored_by', 'src'), ('published_in', 'src'),
              ('discusses', 'src'), ('cites', 'dst')],
    'author': [('authored_by', 'dst')],
    'venue': [('published_in', 'dst')],
    'topic': [('discusses', 'dst')],
}
_REL_FULLKEY = {'cites': 'paper__cites__paper',
                'authored_by': 'paper__authored_by__author',
                'published_in': 'paper__published_in__venue',
                'discusses': 'paper__discusses__topic'}


_MB3 = 128
_CH = _MB3 * 8 // 128
_NDC = _MB3 // 128


def _epad(e):
    blk = _NW * _MB3
    return ((e + blk - 1) // blk) * blk


def _tc_proj(n_rows, n_roles, merge, relu_in, n_rows_pad=0):
    bn = 1000
    grid = n_rows // bn
    w16 = 16 * n_roles

    def body(*refs):
        i = pl.program_id(0)
        if merge:
            o0, o1, w, b = refs[:4]
            x = o0[...] + o1[...]
            if relu_in:
                x = jnp.maximum(x, 0.0)
        else:
            xr, w, b = refs[:3]
            x = xr[...]
        nin = 4 if merge else 3
        xp = jnp.dot(x, w[...], preferred_element_type=jnp.float32) + b[...][0]
        if n_roles:
            lin = refs[nin]
            xp_ref = refs[nin + 1]
            a_refs = refs[nin + 2:nin + 2 + n_roles]
            amax_ref = refs[nin + 2 + n_roles]
            xp_ref[...] = xp
            av = jnp.dot(xp, lin[...], preferred_element_type=jnp.float32)
            for r in range(n_roles):
                a_refs[r][...] = av[:, 16 * r:16 * r + 16]
            cur = jnp.broadcast_to(jnp.max(av, axis=0, keepdims=True),
                                   (8, w16))

            @pl.when(i == 0)
            def _():
                amax_ref[...] = cur

            @pl.when(i != 0)
            def _():
                amax_ref[...] = jnp.maximum(amax_ref[...], cur)
        else:
            xp_ref = refs[nin]
            xp_ref[...] = xp

    row_spec = pl.BlockSpec((bn, _HID), lambda i: (i, 0))
    full = lambda shape: pl.BlockSpec(shape, lambda i: (0, 0))
    in_specs = ([row_spec, row_spec] if merge else [row_spec])
    in_specs += [full((_HID, _HID)), full((8, _HID))]
    out_shapes = [jax.ShapeDtypeStruct((n_rows, _HID), jnp.float32)]
    out_specs = [row_spec]
    if n_roles:
        in_specs += [full((_HID, w16))]
        out_shapes += [jax.ShapeDtypeStruct((n_rows_pad, 16), jnp.float32)
                       for _ in range(n_roles)]
        out_specs += [pl.BlockSpec((bn, 16), lambda i: (i, 0))
                      for _ in range(n_roles)]
        out_shapes += [jax.ShapeDtypeStruct((8, w16), jnp.float32)]
        out_specs += [full((8, w16))]

    return pl.pallas_call(
        body, grid=(grid,), in_specs=in_specs,
        out_specs=out_specs, out_shape=out_shapes)


_SC_PARAMS = pltpu.CompilerParams(use_tc_tiling_on_sc=False)

_MESH = functools.partial(plsc.VectorSubcoreMesh,
                          core_axis_name='c', subcore_axis_name='s',
                          num_cores=_NC, num_subcores=_NS)


def _leaky(s):
    return jnp.maximum(s, 0.0) + 0.2 * jnp.minimum(s, 0.0)


def _sc_p1(epads, e_reals, npads):
    nrel = len(_RELS)

    out_type = []
    for r in range(nrel):
        out_type.append(jax.ShapeDtypeStruct((epads[r], 16), jnp.float32))
        out_type.append(jax.ShapeDtypeStruct((npads[r], 16), jnp.float32))
        out_type.append(jax.ShapeDtypeStruct((npads[r], 16), jnp.float32))

    scratch = [
        pltpu.VMEM((_MB,), jnp.int32),
        pltpu.VMEM((_MB,), jnp.int32),
        pltpu.VMEM((_MB, 16), jnp.float32),
        pltpu.VMEM((_MB, 16), jnp.float32),
        pltpu.VMEM((_MB, 16), jnp.float32),
        pltpu.VMEM((16,), jnp.float32),
        pltpu.SemaphoreType.DMA,
    ] + [pltpu.VMEM_SHARED((npads[r], 16), jnp.float32) for r in range(nrel)]

    def body(*refs):
        zero_den = refs[0]
        ins = refs[1:1 + 5 * nrel]
        outs = refs[1 + 5 * nrel:1 + 8 * nrel]
        (src_v, dst_v, asrc_v, adst_v, ex_v, mv, sem) = refs[1 + 8 * nrel:
                                                             8 + 8 * nrel]
        den_sps = refs[8 + 8 * nrel:]
        cid = lax.axis_index('c')
        sid = lax.axis_index('s')
        wid = sid * _NC + cid

        for r in range(nrel):
            src_h, dst_h, a_src_h, a_dst_h, mvec_h = ins[5 * r:5 * r + 5]
            ex_h, den0_h, den1_h = outs[3 * r:3 * r + 3]
            den_sp = den_sps[r]
            npad, epad, ereal = npads[r], epads[r], e_reals[r]
            rows = npad // _NS
            pltpu.sync_copy(zero_den.at[pl.ds(0, rows)],
                            den_sp.at[pl.ds(sid * rows, rows)])
            plsc.subcore_barrier()
            pltpu.sync_copy(mvec_h, mv)
            m16 = mv[...]
            ew = epad // _NW
            nb = ew // _MB
            base = wid * ew

            def batch(t, carry):
                off = base + t * _MB
                pltpu.sync_copy(src_h.at[pl.ds(off, _MB)], src_v)
                pltpu.sync_copy(dst_h.at[pl.ds(off, _MB)], dst_v)
                cpa = pltpu.async_copy(a_src_h.at[src_v], asrc_v, sem)
                cpb = pltpu.async_copy(a_dst_h.at[dst_v], adst_v, sem)
                cpa.wait()
                cpb.wait()
                for j in range(_MB):
                    s = asrc_v[j, :] + adst_v[j, :]
                    ex_v[j, :] = jnp.exp(_leaky(s) - m16)
                pltpu.sync_copy(ex_v, ex_h.at[pl.ds(off, _MB)])
                pltpu.sync_copy(ex_v, den_sp.at[dst_v], add=True)
                return carry

            lax.fori_loop(0, nb, batch, 0)
            plsc.subcore_barrier()

            @pl.when(cid == 0)
            def _():
                pltpu.sync_copy(den_sp.at[pl.ds(sid * rows, rows)],
                                den0_h.at[pl.ds(sid * rows, rows)])

            @pl.when(cid == 1)
            def _():
                pltpu.sync_copy(den_sp.at[pl.ds(sid * rows, rows)],
                                den1_h.at[pl.ds(sid * rows, rows)])
            plsc.subcore_barrier()

    return pl.kernel(body, out_type=tuple(out_type), mesh=_MESH(),
                     scratch_types=scratch, compiler_params=_SC_PARAMS)


def _sc_p3(epads, npads):
    nrel = len(_RELS)
    out_type = []
    nbks = []
    for r in range(nrel):
        out_type.append(jax.ShapeDtypeStruct((npads[r] * 8, 16), jnp.float32))
        out_type.append(jax.ShapeDtypeStruct((npads[r] * 8, 16), jnp.float32))
        nbks.append(npads[r] // _BUCKET if npads[r] >= _BUCKET else 1)

    scratch = (
        [pltpu.VMEM((128,), jnp.int32) for _ in range(_NDC)]
        + [pltpu.VMEM((128,), jnp.int32) for _ in range(_CH)]
        + [pltpu.VMEM((128,), jnp.int32) for _ in range(_CH)]
        + [pltpu.VMEM((_MB3, 16), jnp.float32),
           pltpu.VMEM((_MB3, 16), jnp.float32),
           pltpu.VMEM((_MB3, 16), jnp.float32),
           pltpu.VMEM((_MB3, 16), jnp.float32),
           pltpu.VMEM((_MB3 * 8, 16), jnp.float32),
           pltpu.SemaphoreType.DMA,
           pltpu.SemaphoreType.DMA,
           pltpu.VMEM_SHARED(((_BUCKET + 1) * 8, 16), jnp.float32)])
    n_in = [5 + nbks[r] for r in range(nrel)]

    def body(*refs):
        zero_acc = refs[0]
        xp2 = refs[1]
        ins = refs[2:2 + sum(n_in)]
        outs = refs[2 + sum(n_in):2 + sum(n_in) + 2 * nrel]
        sc_refs = refs[2 + sum(n_in) + 2 * nrel:]
        dst_vs = sc_refs[0:_NDC]
        eidx_vs = sc_refs[_NDC:_NDC + _CH]
        didx_vs = sc_refs[_NDC + _CH:_NDC + 2 * _CH]
        (ex_v, d0_v, d1_v, w_v, xp_v, sem, sem2,
         acc) = sc_refs[_NDC + 2 * _CH:]
        cid = lax.axis_index('c')
        sid = lax.axis_index('s')
        wid = sid * _NC + cid

        ioff = 0
        for r in range(nrel):
            dst_h, ex_h, den0_h, den1_h, eidx_h = ins[ioff:ioff + 5]
            didx_hs = ins[ioff + 5:ioff + n_in[r]]
            ioff += n_in[r]
            o0_h, o1_h = outs[2 * r:2 * r + 2]
            npad, epad = npads[r], epads[r]
            nbuckets = nbks[r]
            brows = npad // nbuckets
            rows_pb = brows // _NS
            ew = epad // _NW
            nb = ew // _MB3
            base = wid * ew

            for bkt in range(nbuckets):
                lo = bkt * brows
                didx_h = didx_hs[bkt]
                pltpu.sync_copy(zero_acc.at[pl.ds(0, rows_pb * 8)],
                                acc.at[pl.ds(sid * rows_pb * 8, rows_pb * 8)])
                plsc.subcore_barrier()

                def batch(t, carry):
                    off = base + t * _MB3
                    for k in range(_NDC):
                        pltpu.sync_copy(dst_h.at[pl.ds(off + 128 * k, 128)],
                                        dst_vs[k])
                    pltpu.sync_copy(ex_h.at[pl.ds(off, _MB3)], ex_v)
                    for k in range(_CH):
                        pltpu.sync_copy(eidx_h.at[off // 16 + k], eidx_vs[k])
                        pltpu.sync_copy(didx_h.at[off // 16 + k], didx_vs[k])
                    cpds = [pltpu.async_copy(
                        den0_h.at[dst_vs[k]],
                        d0_v.at[pl.ds(128 * k, 128)], sem2)
                        for k in range(_NDC)]
                    cpds += [pltpu.async_copy(
                        den1_h.at[dst_vs[k]],
                        d1_v.at[pl.ds(128 * k, 128)], sem2)
                        for k in range(_NDC)]
                    cps = [pltpu.async_copy(xp2.at[eidx_vs[k]],
                                            xp_v.at[pl.ds(128 * k, 128)],
                                            sem)
                           for k in range(_CH)]
                    for cp in cpds:
                        cp.wait()

                    def wrow(j, cw):
                        w_v[j, :] = ex_v[j, :] / (d0_v[j, :] + d1_v[j, :]
                                                  + 1e-16)
                        return cw

                    lax.fori_loop(0, _MB3, wrow, 0)
                    for cp in cps:
                        cp.wait()

                    def edge(e2, c2):
                        for ei in range(2):
                            e = 2 * e2 + ei
                            w16 = w_v[e, :]
                            r0 = 8 * e
                            for h in range(4):
                                wb = w16.at[jnp.full((16,), h,
                                                     jnp.int32)].get(
                                    mode='promise_in_bounds')
                                for j in (2 * h, 2 * h + 1):
                                    xp_v[r0 + j, :] = xp_v[r0 + j, :] * wb
                        return c2

                    lax.fori_loop(0, _MB3 // 2, edge, 0)
                    for k0 in range(0, _CH, 8):
                        scs = [pltpu.async_copy(
                            xp_v.at[pl.ds(128 * k, 128)],
                            acc.at[didx_vs[k]], sem2, add=True)
                            for k in range(k0, k0 + 8)]
                        for sc in scs:
                            sc.wait()
                    return carry

                lax.fori_loop(0, nb, batch, 0)
                plsc.subcore_barrier()

                @pl.when(cid == 0)
                def _():
                    pltpu.sync_copy(
                        acc.at[pl.ds(sid * rows_pb * 8, rows_pb * 8)],
                        o0_h.at[pl.ds((lo + sid * rows_pb) * 8,
                                      rows_pb * 8)])

                @pl.when(cid == 1)
                def _():
                    pltpu.sync_copy(
                        acc.at[pl.ds(sid * rows_pb * 8, rows_pb * 8)],
                        o1_h.at[pl.ds((lo + sid * rows_pb) * 8,
                                      rows_pb * 8)])
                plsc.subcore_barrier()

    return pl.kernel(body, out_type=tuple(out_type), mesh=_MESH(),
                     scratch_types=scratch, compiler_params=_SC_PARAMS)


def _build_lin(lp):
    lins = {}
    for nt in _NTYPES:
        roles = _ROLES[nt]
        cols = jnp.zeros((_HID, 16 * len(roles)), jnp.float32)
        for ridx, (rel, role) in enumerate(roles):
            key = _REL_FULLKEY[rel]
            vec = lp['lin_src'][key] if role == 'src' else lp['lin_dst'][key]
            for h in range(_HEADS):
                cols = cols.at[32 * h:32 * h + 32, 16 * ridx + h].set(vec[h])
        lins[nt] = cols
    return lins


def kernel(x_paper, x_author, x_venue, x_topic, edge_index_cites,
           edge_index_authored_by, edge_index_published_in,
           edge_index_discusses, params):
    x_in = {'paper': x_paper, 'author': x_author, 'venue': x_venue,
            'topic': x_topic}
    e_idx = {'cites': edge_index_cites, 'authored_by': edge_index_authored_by,
             'published_in': edge_index_published_in,
             'discusses': edge_index_discusses}

    epads = [_epad(e) for (_, _, _, e) in _RELS]
    e_reals = [e for (_, _, _, e) in _RELS]
    npads = [_NPAD[dt] for (_, _, dt, _) in _RELS]
    srcs, dsts = [], []
    for ri, (rel, st, dt, e) in enumerate(_RELS):
        ei = e_idx[rel]
        pad = epads[ri] - e
        srcs.append(jnp.pad(ei[0], (0, pad)))
        dsts.append(jnp.pad(ei[1], (0, pad), constant_values=npads[ri] - 1))

    zero_den = jnp.zeros((_NPAD['paper'] // _NS, 16), jnp.float32)
    zero_acc = jnp.zeros((_BUCKET // _NS * 8, 16), jnp.float32)

    j8 = jnp.arange(8, dtype=jnp.int32)
    eidx8s, didx8s = [], []
    for ri in range(len(_RELS)):
        npad = npads[ri]
        eidx8s.append((8 * srcs[ri][:, None] + j8).reshape(-1, 128))
        nbk = npad // _BUCKET if npad >= _BUCKET else 1
        brows = npad // nbk
        dlist = []
        for bkt in range(nbk):
            rel = dsts[ri] - bkt * brows
            rel = jnp.where((rel >= 0) & (rel < brows), rel, _BUCKET)
            dlist.append((8 * rel[:, None] + j8).reshape(-1, 128))
        didx8s.append(dlist)

    p1 = _sc_p1(epads, e_reals, npads)
    p3 = _sc_p3(epads, npads)

    prev = None
    for lp in params['layers']:
        lins = _build_lin(lp)
        xp, a_arrs, amax = {}, {}, {}
        for nt in _NTYPES:
            n_roles = len(_ROLES[nt])
            tc = _tc_proj(_NN[nt], n_roles, merge=prev is not None,
                          relu_in=True, n_rows_pad=_NPAD[nt])
            b8 = jnp.tile(lp['proj_b'][nt][None, :], (8, 1))
            if prev is None:
                res = tc(x_in[nt], lp['proj_w'][nt], b8, lins[nt])
            else:
                o0, o1 = prev[nt]
                res = tc(o0, o1, lp['proj_w'][nt], b8, lins[nt])
            xp[nt] = res[0]
            a_arrs[nt] = res[1:1 + n_roles]
            amax[nt] = res[1 + n_roles]

        p1_ins = [zero_den]
        for ri, (rel, st, dt, e) in enumerate(_RELS):
            src_role = _ROLES[st].index((rel, 'src'))
            dst_role = _ROLES[dt].index((rel, 'dst'))
            a_src = a_arrs[st][src_role]
            a_dst = a_arrs[dt][dst_role]
            msrc = amax[st][0, 16 * src_role:16 * src_role + 4]
            mdst = amax[dt][0, 16 * dst_role:16 * dst_role + 4]
            m4 = _leaky(msrc + mdst)
            mvec = jnp.concatenate([m4, jnp.full((12,), 1e30, jnp.float32)])
            p1_ins += [srcs[ri], dsts[ri], a_src, a_dst, mvec]
        p1_out = p1(*p1_ins)

        p3_ins = [zero_acc, xp['paper'].reshape(_NN['paper'] * 8, 16)]
        for ri in range(len(_RELS)):
            ex, den0, den1 = p1_out[3 * ri:3 * ri + 3]
            p3_ins += [dsts[ri], ex, den0, den1, eidx8s[ri]] + didx8s[ri]
        p3_out = p3(*p3_ins)

        prev = {}
        for ri, (rel, st, dt, e) in enumerate(_RELS):
            npad = npads[ri]
            prev[dt] = (p3_out[2 * ri].reshape(npad, _HID),
                        p3_out[2 * ri + 1].reshape(npad, _HID))

    outs = []
    for nt in _NTYPES:
        tc = _tc_proj(_NN[nt], 0, merge=True, relu_in=True)
        b8 = jnp.tile(params['out_b'][nt][None, :], (8, 1))
        o0, o1 = prev[nt]
        outs.append(tc(o0, o1, params['out_w'][nt], b8)[0])
    return tuple(outs)


# speedup vs baseline: 10.6137x; 1.0199x over previous
import functools
import jax
import jax.numpy as jnp
from jax import lax
from jax.experimental import pallas as pl
from jax.experimental.pallas import tpu as pltpu
from jax.experimental.pallas import tpu_sc as plsc

_HID = 128
_HEADS = 4
_NTYPES = ('paper', 'author', 'venue', 'topic')
_NN = {'paper': 50000, 'author': 50000, 'venue': 10000, 'topic': 10000}
_RELS = (('cites', 'paper', 'paper', 400000),
         ('authored_by', 'paper', 'author', 200000),
         ('published_in', 'paper', 'venue', 50000),
         ('discusses', 'paper', 'topic', 100000))
_NC, _NS = 2, 16
_NW = _NC * _NS
_MB = 128
_NPAD = {'paper': 51200, 'author': 51200, 'venue': 10240, 'topic': 10240}
_BUCKET = 12800
_ROLES = {
    'paper': [('cites', 'src'), ('authored_by', 'src'), ('published_in', 'src'),
              ('discusses', 'src'), ('cites', 'dst')],
    'author': [('authored_by', 'dst')],
    'venue': [('published_in', 'dst')],
    'topic': [('discusses', 'dst')],
}
_REL_FULLKEY = {'cites': 'paper__cites__paper',
                'authored_by': 'paper__authored_by__author',
                'published_in': 'paper__published_in__venue',
                'discusses': 'paper__discusses__topic'}


_MB3 = 128
_CH = _MB3 * 8 // 128
_NDC = _MB3 // 128


def _epad(e):
    blk = _NW * _MB3
    return ((e + blk - 1) // blk) * blk


def _tc_proj(n_rows, n_roles, merge, relu_in, n_rows_pad=0):
    bn = 1000
    grid = n_rows // bn
    w16 = 16 * n_roles

    def body(*refs):
        i = pl.program_id(0)
        if merge:
            o0, o1, w, b = refs[:4]
            x = o0[...] + o1[...]
            if relu_in:
                x = jnp.maximum(x, 0.0)
        else:
            xr, w, b = refs[:3]
            x = xr[...]
        nin = 4 if merge else 3
        xp = jnp.dot(x, w[...], preferred_element_type=jnp.float32) + b[...][0]
        if n_roles:
            lin = refs[nin]
            xp_ref = refs[nin + 1]
            a_refs = refs[nin + 2:nin + 2 + n_roles]
            amax_ref = refs[nin + 2 + n_roles]
            xp_ref[...] = xp
            av = jnp.dot(xp, lin[...], preferred_element_type=jnp.float32)
            for r in range(n_roles):
                a_refs[r][...] = av[:, 16 * r:16 * r + 16]
            cur = jnp.broadcast_to(jnp.max(av, axis=0, keepdims=True),
                                   (8, w16))

            @pl.when(i == 0)
            def _():
                amax_ref[...] = cur

            @pl.when(i != 0)
            def _():
                amax_ref[...] = jnp.maximum(amax_ref[...], cur)
        else:
            xp_ref = refs[nin]
            xp_ref[...] = xp

    row_spec = pl.BlockSpec((bn, _HID), lambda i: (i, 0))
    full = lambda shape: pl.BlockSpec(shape, lambda i: (0, 0))
    in_specs = ([row_spec, row_spec] if merge else [row_spec])
    in_specs += [full((_HID, _HID)), full((8, _HID))]
    out_shapes = [jax.ShapeDtypeStruct((n_rows, _HID), jnp.float32)]
    out_specs = [row_spec]
    if n_roles:
        in_specs += [full((_HID, w16))]
        out_shapes += [jax.ShapeDtypeStruct((n_rows_pad, 16), jnp.float32)
                       for _ in range(n_roles)]
        out_specs += [pl.BlockSpec((bn, 16), lambda i: (i, 0))
                      for _ in range(n_roles)]
        out_shapes += [jax.ShapeDtypeStruct((8, w16), jnp.float32)]
        out_specs += [full((8, w16))]

    return pl.pallas_call(
        body, grid=(grid,), in_specs=in_specs,
        out_specs=out_specs, out_shape=out_shapes)


_SC_PARAMS = pltpu.CompilerParams(use_tc_tiling_on_sc=False)

_MESH = functools.partial(plsc.VectorSubcoreMesh,
                          core_axis_name='c', subcore_axis_name='s',
                          num_cores=_NC, num_subcores=_NS)


def _leaky(s):
    return jnp.maximum(s, 0.0) + 0.2 * jnp.minimum(s, 0.0)


def _sc_p1(epads, e_reals, npads):
    nrel = len(_RELS)

    out_type = []
    for r in range(nrel):
        out_type.append(jax.ShapeDtypeStruct((epads[r], 16), jnp.float32))
        out_type.append(jax.ShapeDtypeStruct((npads[r], 16), jnp.float32))
        out_type.append(jax.ShapeDtypeStruct((npads[r], 16), jnp.float32))

    scratch = [
        pltpu.VMEM((_MB,), jnp.int32),
        pltpu.VMEM((_MB,), jnp.int32),
        pltpu.VMEM((_MB, 16), jnp.float32),
        pltpu.VMEM((_MB, 16), jnp.float32),
        pltpu.VMEM((_MB, 16), jnp.float32),
        pltpu.VMEM((16,), jnp.float32),
        pltpu.SemaphoreType.DMA,
    ] + [pltpu.VMEM_SHARED((npads[r], 16), jnp.float32) for r in range(nrel)]

    def body(*refs):
        zero_den = refs[0]
        ins = refs[1:1 + 5 * nrel]
        outs = refs[1 + 5 * nrel:1 + 8 * nrel]
        (src_v, dst_v, asrc_v, adst_v, ex_v, mv, sem) = refs[1 + 8 * nrel:
                                                             8 + 8 * nrel]
        den_sps = refs[8 + 8 * nrel:]
        cid = lax.axis_index('c')
        sid = lax.axis_index('s')
        wid = sid * _NC + cid

        for r in range(nrel):
            src_h, dst_h, a_src_h, a_dst_h, mvec_h = ins[5 * r:5 * r + 5]
            ex_h, den0_h, den1_h = outs[3 * r:3 * r + 3]
            den_sp = den_sps[r]
            npad, epad, ereal = npads[r], epads[r], e_reals[r]
            rows = npad // _NS
            pltpu.sync_copy(zero_den.at[pl.ds(0, rows)],
                            den_sp.at[pl.ds(sid * rows, rows)])
            plsc.subcore_barrier()
            pltpu.sync_copy(mvec_h, mv)
            m16 = mv[...]
            ew = epad // _NW
            nb = ew // _MB
            base = wid * ew

            def batch(t, carry):
                off = base + t * _MB
                pltpu.sync_copy(src_h.at[pl.ds(off, _MB)], src_v)
                pltpu.sync_copy(dst_h.at[pl.ds(off, _MB)], dst_v)
                cpa = pltpu.async_copy(a_src_h.at[src_v], asrc_v, sem)
                cpb = pltpu.async_copy(a_dst_h.at[dst_v], adst_v, sem)
                cpa.wait()
                cpb.wait()
                for j in range(_MB):
                    s = asrc_v[j, :] + adst_v[j, :]
                    ex_v[j, :] = jnp.exp(_leaky(s) - m16)
                pltpu.sync_copy(ex_v, ex_h.at[pl.ds(off, _MB)])
                pltpu.sync_copy(ex_v, den_sp.at[dst_v], add=True)
                return carry

            lax.fori_loop(0, nb, batch, 0)
            plsc.subcore_barrier()

            @pl.when(cid == 0)
            def _():
                pltpu.sync_copy(den_sp.at[pl.ds(sid * rows, rows)],
                                den0_h.at[pl.ds(sid * rows, rows)])

            @pl.when(cid == 1)
            def _():
                pltpu.sync_copy(den_sp.at[pl.ds(sid * rows, rows)],
                                den1_h.at[pl.ds(sid * rows, rows)])
            plsc.subcore_barrier()

    return pl.kernel(body, out_type=tuple(out_type), mesh=_MESH(),
                     scratch_types=scratch, compiler_params=_SC_PARAMS)


def _sc_p3(epads, npads):
    nrel = len(_RELS)
    out_type = []
    nbks = []
    for r in range(nrel):
        out_type.append(jax.ShapeDtypeStruct((npads[r] * 8, 16), jnp.float32))
        out_type.append(jax.ShapeDtypeStruct((npads[r] * 8, 16), jnp.float32))
        out_type.append(jax.ShapeDtypeStruct((epads[r], 16), jnp.float32))
        nbks.append(npads[r] // _BUCKET if npads[r] >= _BUCKET else 1)

    scratch = (
        [pltpu.VMEM((128,), jnp.int32) for _ in range(_NDC)]
        + [pltpu.VMEM((128,), jnp.int32) for _ in range(_CH)]
        + [pltpu.VMEM((128,), jnp.int32) for _ in range(_CH)]
        + [pltpu.VMEM((_MB3, 16), jnp.float32),
           pltpu.VMEM((_MB3, 16), jnp.float32),
           pltpu.VMEM((_MB3, 16), jnp.float32),
           pltpu.VMEM((_MB3, 16), jnp.float32),
           pltpu.VMEM((_MB3 * 8, 16), jnp.float32),
           pltpu.SemaphoreType.DMA,
           pltpu.SemaphoreType.DMA,
           pltpu.VMEM_SHARED(((_BUCKET + 1) * 8, 16), jnp.float32)])
    n_in = [5 + nbks[r] for r in range(nrel)]

    def body(*refs):
        zero_acc = refs[0]
        xp2 = refs[1]
        ins = refs[2:2 + sum(n_in)]
        outs = refs[2 + sum(n_in):2 + sum(n_in) + 3 * nrel]
        sc_refs = refs[2 + sum(n_in) + 3 * nrel:]
        dst_vs = sc_refs[0:_NDC]
        eidx_vs = sc_refs[_NDC:_NDC + _CH]
        didx_vs = sc_refs[_NDC + _CH:_NDC + 2 * _CH]
        (ex_v, d0_v, d1_v, w_v, xp_v, sem, sem2,
         acc) = sc_refs[_NDC + 2 * _CH:]
        cid = lax.axis_index('c')
        sid = lax.axis_index('s')
        wid = sid * _NC + cid

        ioff = 0
        for r in range(nrel):
            dst_h, ex_h, den0_h, den1_h, eidx_h = ins[ioff:ioff + 5]
            didx_hs = ins[ioff + 5:ioff + n_in[r]]
            ioff += n_in[r]
            o0_h, o1_h, w_h = outs[3 * r:3 * r + 3]
            npad, epad = npads[r], epads[r]
            nbuckets = nbks[r]
            brows = npad // nbuckets
            rows_pb = brows // _NS
            ew = epad // _NW
            nb = ew // _MB3
            base = wid * ew

            for bkt in range(nbuckets):
                lo = bkt * brows
                didx_h = didx_hs[bkt]
                pltpu.sync_copy(zero_acc.at[pl.ds(0, rows_pb * 8)],
                                acc.at[pl.ds(sid * rows_pb * 8, rows_pb * 8)])
                plsc.subcore_barrier()

                def batch(t, carry):
                    off = base + t * _MB3
                    if bkt == 0:
                        for k in range(_NDC):
                            pltpu.sync_copy(
                                dst_h.at[pl.ds(off + 128 * k, 128)],
                                dst_vs[k])
                        pltpu.sync_copy(ex_h.at[pl.ds(off, _MB3)], ex_v)
                    for k in range(_CH):
                        pltpu.sync_copy(eidx_h.at[off // 16 + k], eidx_vs[k])
                        pltpu.sync_copy(didx_h.at[off // 16 + k], didx_vs[k])
                    if bkt == 0:
                        cpds = [pltpu.async_copy(
                            den0_h.at[dst_vs[k]],
                            d0_v.at[pl.ds(128 * k, 128)], sem2)
                            for k in range(_NDC)]
                        cpds += [pltpu.async_copy(
                            den1_h.at[dst_vs[k]],
                            d1_v.at[pl.ds(128 * k, 128)], sem2)
                            for k in range(_NDC)]
                    else:
                        pltpu.sync_copy(w_h.at[pl.ds(off, _MB3)], w_v)
                    cps = [pltpu.async_copy(xp2.at[eidx_vs[k]],
                                            xp_v.at[pl.ds(128 * k, 128)],
                                            sem)
                           for k in range(_CH)]
                    if bkt == 0:
                        for cp in cpds:
                            cp.wait()

                        def wrow(j, cw):
                            w_v[j, :] = ex_v[j, :] / (d0_v[j, :] + d1_v[j, :]
                                                      + 1e-16)
                            return cw

                        lax.fori_loop(0, _MB3, wrow, 0)
                        pltpu.sync_copy(w_v, w_h.at[pl.ds(off, _MB3)])
                    for cp in cps:
                        cp.wait()

                    def edge(e2, c2):
                        for ei in range(2):
                            e = 2 * e2 + ei
                            w16 = w_v[e, :]
                            r0 = 8 * e
                            for h in range(4):
                                wb = w16.at[jnp.full((16,), h,
                                                     jnp.int32)].get(
                                    mode='promise_in_bounds')
                                for j in (2 * h, 2 * h + 1):
                                    xp_v[r0 + j, :] = xp_v[r0 + j, :] * wb
                        return c2

                    lax.fori_loop(0, _MB3 // 2, edge, 0)
                    for k0 in range(0, _CH, 8):
                        scs = [pltpu.async_copy(
                            xp_v.at[pl.ds(128 * k, 128)],
                            acc.at[didx_vs[k]], sem2, add=True)
                            for k in range(k0, k0 + 8)]
                        for sc in scs:
                            sc.wait()
                    return carry

                lax.fori_loop(0, nb, batch, 0)
                plsc.subcore_barrier()

                @pl.when(cid == 0)
                def _():
                    pltpu.sync_copy(
                        acc.at[pl.ds(sid * rows_pb * 8, rows_pb * 8)],
                        o0_h.at[pl.ds((lo + sid * rows_pb) * 8,
                                      rows_pb * 8)])

                @pl.when(cid == 1)
                def _():
                    pltpu.sync_copy(
                        acc.at[pl.ds(sid * rows_pb * 8, rows_pb * 8)],
                        o1_h.at[pl.ds((lo + sid * rows_pb) * 8,
                                      rows_pb * 8)])
                plsc.subcore_barrier()

    return pl.kernel(body, out_type=tuple(out_type), mesh=_MESH(),
                     scratch_types=scratch, compiler_params=_SC_PARAMS)


def _build_lin(lp):
    lins = {}
    for nt in _NTYPES:
        roles = _ROLES[nt]
        cols = jnp.zeros((_HID, 16 * len(roles)), jnp.float32)
        for ridx, (rel, role) in enumerate(roles):
            key = _REL_FULLKEY[rel]
            vec = lp['lin_src'][key] if role == 'src' else lp['lin_dst'][key]
            for h in range(_HEADS):
                cols = cols.at[32 * h:32 * h + 32, 16 * ridx + h].set(vec[h])
        lins[nt] = cols
    return lins


def kernel(x_paper, x_author, x_venue, x_topic, edge_index_cites,
           edge_index_authored_by, edge_index_published_in,
           edge_index_discusses, params):
    x_in = {'paper': x_paper, 'author': x_author, 'venue': x_venue,
            'topic': x_topic}
    e_idx = {'cites': edge_index_cites, 'authored_by': edge_index_authored_by,
             'published_in': edge_index_published_in,
             'discusses': edge_index_discusses}

    epads = [_epad(e) for (_, _, _, e) in _RELS]
    e_reals = [e for (_, _, _, e) in _RELS]
    npads = [_NPAD[dt] for (_, _, dt, _) in _RELS]
    srcs, dsts = [], []
    for ri, (rel, st, dt, e) in enumerate(_RELS):
        ei = e_idx[rel]
        pad = epads[ri] - e
        srcs.append(jnp.pad(ei[0], (0, pad)))
        dsts.append(jnp.pad(ei[1], (0, pad), constant_values=npads[ri] - 1))

    zero_den = jnp.zeros((_NPAD['paper'] // _NS, 16), jnp.float32)
    zero_acc = jnp.zeros((_BUCKET // _NS * 8, 16), jnp.float32)

    j8 = jnp.arange(8, dtype=jnp.int32)
    eidx8s, didx8s = [], []
    for ri in range(len(_RELS)):
        npad = npads[ri]
        eidx8s.append((8 * srcs[ri][:, None] + j8).reshape(-1, 128))
        nbk = npad // _BUCKET if npad >= _BUCKET else 1
        brows = npad // nbk
        dlist = []
        for bkt in range(nbk):
            rel = dsts[ri] - bkt * brows
            rel = jnp.where((rel >= 0) & (rel < brows), rel, _BUCKET)
            dlist.append((8 * rel[:, None] + j8).reshape(-1, 128))
        didx8s.append(dlist)

    p1 = _sc_p1(epads, e_reals, npads)
    p3 = _sc_p3(epads, npads)

    prev = None
    for lp in params['layers']:
        lins = _build_lin(lp)
        xp, a_arrs, amax = {}, {}, {}
        for nt in _NTYPES:
            n_roles = len(_ROLES[nt])
            tc = _tc_proj(_NN[nt], n_roles, merge=prev is not None,
                          relu_in=True, n_rows_pad=_NPAD[nt])
            b8 = jnp.tile(lp['proj_b'][nt][None, :], (8, 1))
            if prev is None:
                res = tc(x_in[nt], lp['proj_w'][nt], b8, lins[nt])
            else:
                o0, o1 = prev[nt]
                res = tc(o0, o1, lp['proj_w'][nt], b8, lins[nt])
            xp[nt] = res[0]
            a_arrs[nt] = res[1:1 + n_roles]
            amax[nt] = res[1 + n_roles]

        p1_ins = [zero_den]
        for ri, (rel, st, dt, e) in enumerate(_RELS):
            src_role = _ROLES[st].index((rel, 'src'))
            dst_role = _ROLES[dt].index((rel, 'dst'))
            a_src = a_arrs[st][src_role]
            a_dst = a_arrs[dt][dst_role]
            msrc = amax[st][0, 16 * src_role:16 * src_role + 4]
            mdst = amax[dt][0, 16 * dst_role:16 * dst_role + 4]
            m4 = _leaky(msrc + mdst)
            mvec = jnp.concatenate([m4, jnp.full((12,), 1e30, jnp.float32)])
            p1_ins += [srcs[ri], dsts[ri], a_src, a_dst, mvec]
        p1_out = p1(*p1_ins)

        p3_ins = [zero_acc, xp['paper'].reshape(_NN['paper'] * 8, 16)]
        for ri in range(len(_RELS)):
            ex, den0, den1 = p1_out[3 * ri:3 * ri + 3]
            p3_ins += [dsts[ri], ex, den0, den1, eidx8s[ri]] + didx8s[ri]
        p3_out = p3(*p3_ins)

        prev = {}
        for ri, (rel, st, dt, e) in enumerate(_RELS):
            npad = npads[ri]
            prev[dt] = (p3_out[3 * ri].reshape(npad, _HID),
                        p3_out[3 * ri + 1].reshape(npad, _HID))

    outs = []
    for nt in _NTYPES:
        tc = _tc_proj(_NN[nt], 0, merge=True, relu_in=True)
        b8 = jnp.tile(params['out_b'][nt][None, :], (8, 1))
        o0, o1 = prev[nt]
        outs.append(tc(o0, o1, params['out_w'][nt], b8)[0])
    return tuple(outs)
